# Initial kernel scaffold; baseline (speedup 1.0000x reference)
import jax
import jax.numpy as jnp
from jax.experimental import pallas as pl


def kernel(x, edge_index, edge_attr, mask, batch, params):
    raise NotImplementedError("write your pallas kernel here")


# speedup vs baseline: 46.0019x; 46.0019x over previous
import functools

import jax
import jax.numpy as jnp
from jax import lax
from jax.experimental import pallas as pl
from jax.experimental.pallas import tpu as pltpu
from jax.experimental.pallas import tpu_sc as plsc

_N = 10000
_E = 640000
_HEADS = 16
_NG = 64
_B = 128
_NW = 32
_ETOT = _E + _N
_CHUNKS = -(-_ETOT // (_NW * _B))
_EPAD = _CHUNKS * _NW * _B
_PERW = _CHUNKS * _B
_RPT = 632
_ACC_ROWS = _RPT * 16
_TAB = 10008
_NP = 10112


def _make_edge_kernel(gs):
    num_sub = len(gs)
    offs, o = [], 0
    for g in gs:
        offs.append(o)
        o += g * 16
    sum_c = o
    w_cols = sum_c + 16 * num_sub
    mesh = plsc.VectorSubcoreMesh(core_axis_name="c", subcore_axis_name="s")

    @functools.partial(
        pl.kernel,
        mesh=mesh,
        compiler_params=pltpu.CompilerParams(use_tc_tiling_on_sc=False),
        out_type=jax.ShapeDtypeStruct((2 * _ACC_ROWS, w_cols), jnp.float32),
        scratch_types=[
            pltpu.VMEM((_B,), jnp.int32),
            pltpu.VMEM((_B,), jnp.int32),
            pltpu.VMEM((_B * 4,), jnp.float32),
            pltpu.VMEM((_B, sum_c), jnp.float32),
            pltpu.VMEM((_B, sum_c), jnp.float32),
            pltpu.VMEM((_B, w_cols), jnp.float32),
            pltpu.VMEM((4 * sum_c,), jnp.float32),
            pltpu.VMEM((sum_c,), jnp.float32),
            pltpu.VMEM((_RPT, w_cols), jnp.float32),
            pltpu.SemaphoreType.DMA,
            pltpu.SemaphoreType.DMA,
            pltpu.VMEM_SHARED((_ACC_ROWS, w_cols), jnp.float32),
        ],
    )
    def edge_kernel(xlt, xrt, srch, dsth, eah, weh, atth, outh,
                    src_v, dst_v, ea_v, xl_v, xr_v, co_v, we_v, att_v, zb_v,
                    sem1, sem2, acc):
        c = lax.axis_index("c")
        s = lax.axis_index("s")
        wid = s * 2 + c
        zero16 = jnp.zeros((16,), jnp.float32)

        def zrow(i, carry):
            for t in range(w_cols // 16):
                zb_v[i, pl.ds(t * 16, 16)] = zero16
            return carry
        lax.fori_loop(0, _RPT, zrow, 0)
        r0 = s * _RPT
        pltpu.sync_copy(zb_v, acc.at[pl.ds(r0, _RPT)])
        plsc.subcore_barrier()

        pltpu.sync_copy(weh, we_v)
        pltpu.sync_copy(atth, att_v)
        wvecs = [[we_v[pl.ds(k * sum_c + t * 16, 16)] for t in range(sum_c // 16)]
                 for k in range(4)]
        avecs = [att_v[pl.ds(t * 16, 16)] for t in range(sum_c // 16)]

        def chunk(j, carry):
            base = wid * _PERW + j * _B
            pltpu.sync_copy(srch.at[pl.ds(base, _B)], src_v)
            pltpu.sync_copy(dsth.at[pl.ds(base, _B)], dst_v)
            pltpu.sync_copy(eah.at[pl.ds(base * 4, _B * 4)], ea_v)
            g1 = pltpu.async_copy(xlt.at[src_v], xl_v, sem1)
            g2 = pltpu.async_copy(xrt.at[dst_v], xr_v, sem2)
            g1.wait()
            g2.wait()

            def edge_quad(q, icarry):
                ea16 = ea_v[pl.ds(q * 16, 16)]
                for r in range(4):
                    i = q * 4 + r
                    eab = [jnp.full((16,), ea16[r * 4 + k], jnp.float32)
                           for k in range(4)]
                    for t in range(num_sub):
                        alpha = zero16
                        xls = []
                        for gi in range(gs[t]):
                            ti = offs[t] // 16 + gi
                            col = offs[t] + gi * 16
                            xlg = xl_v[i, pl.ds(col, 16)]
                            xrg = xr_v[i, pl.ds(col, 16)]
                            e = (eab[0] * wvecs[0][ti] + eab[1] * wvecs[1][ti]
                                 + eab[2] * wvecs[2][ti] + eab[3] * wvecs[3][ti])
                            m = xlg + xrg + e
                            m = jnp.where(m >= 0.0, m, m * 0.2)
                            alpha = alpha + m * avecs[ti]
                            xls.append(xlg)
                        ex = jnp.exp(alpha)
                        co_v[i, pl.ds(sum_c + t * 16, 16)] = ex
                        for gi in range(gs[t]):
                            col = offs[t] + gi * 16
                            co_v[i, pl.ds(col, 16)] = ex * xls[gi]
                return icarry
            lax.fori_loop(0, _B // 4, edge_quad, 0)
            pltpu.sync_copy(co_v, acc.at[dst_v], add=True)
            return carry
        lax.fori_loop(0, _CHUNKS, chunk, 0)

        plsc.subcore_barrier()
        pltpu.sync_copy(acc.at[pl.ds(r0, _RPT)], zb_v)
        pltpu.sync_copy(zb_v, outh.at[pl.ds(c * _ACC_ROWS + r0, _RPT)])

    return edge_kernel


_edge_k4 = _make_edge_kernel((4,))
_edge_k11 = _make_edge_kernel((1, 1))


def _pool_body(oh_ref, sw_ref, o_ref):
    o_ref[...] = jnp.dot(oh_ref[...], sw_ref[...],
                         preferred_element_type=jnp.float32,
                         precision=jax.lax.Precision.HIGHEST)


def _pool(oh, sw):
    return pl.pallas_call(
        _pool_body,
        out_shape=jax.ShapeDtypeStruct((_NG, 128), jnp.float32),
    )(oh, sw)


def _to_t(a, g):
    return a.reshape(-1, _HEADS, g).transpose(0, 2, 1).reshape(-1, g * _HEADS)


def _edge_pass(plist, gs, h, src, dst, ea_flat):
    xls, xrs, wes, atts = [], [], [], []
    for p, g in zip(plist, gs):
        cc = g * 16
        xls.append(_to_t(h @ p["Wl"] + p["bl"], g))
        xrs.append(_to_t(h @ p["Wr"] + p["br"], g))
        wes.append(p["We"].reshape(4, _HEADS, g).transpose(0, 2, 1).reshape(4, cc))
        atts.append(p["att"].transpose(1, 0).reshape(cc))
    sum_c = sum(g * 16 for g in gs)
    pad = jnp.zeros((_TAB - _N, sum_c), jnp.float32)
    xlt = jnp.concatenate([jnp.concatenate(xls, axis=1), pad], axis=0)
    xrt = jnp.concatenate([jnp.concatenate(xrs, axis=1), pad], axis=0)
    we = jnp.concatenate(wes, axis=1).reshape(-1)
    att = jnp.concatenate(atts)
    ek = _edge_k4 if len(gs) == 1 else _edge_k11
    parts = ek(xlt, xrt, src, dst, ea_flat, we, att)
    acc = parts.reshape(2, _ACC_ROWS, -1).sum(0)[:_N]
    outs = []
    for t, g in enumerate(gs):
        cc = g * 16
        off = sum(gg * 16 for gg in gs[:t])
        num = acc[:, off:off + cc]
        den = acc[:, sum_c + t * 16: sum_c + (t + 1) * 16]
        out_t = num / (jnp.tile(den, (1, g)) + 1e-16)
        outs.append(out_t.reshape(_N, g, _HEADS).transpose(0, 2, 1)
                    .reshape(_N, _HEADS * g))
    return outs


def _bn(h, g, b):
    mu = h.mean(0)
    var = h.var(0)
    return (h - mu) / jnp.sqrt(var + 1e-5) * g + b


def kernel(x, edge_index, edge_attr, mask, batch, params):
    src = edge_index[0].astype(jnp.int32)
    dst = edge_index[1].astype(jnp.int32)
    loop = jnp.arange(_N, dtype=jnp.int32)
    padi = jnp.full((_EPAD - _ETOT,), _N, jnp.int32)
    src = jnp.concatenate([src, loop, padi])
    dst = jnp.concatenate([dst, loop, padi])
    ea = jnp.concatenate([
        edge_attr,
        jnp.broadcast_to(edge_attr.mean(0), (_N, 4)),
        jnp.zeros((_EPAD - _ETOT, 4), jnp.float32),
    ]).reshape(-1)
    p = params

    h = _edge_pass([p["init"]], (4,), x, src, dst, ea)[0] + p["init"]["bias"]
    h = jax.nn.leaky_relu(_bn(h, p["bn0_g"], p["bn0_b"]), 0.01)
    h = _edge_pass([p["hid"]], (4,), h, src, dst, ea)[0] + p["hid"]["bias"]
    h = jax.nn.leaky_relu(_bn(h, p["bn1_g"], p["bn1_b"]), 0.01)
    om, osat = _edge_pass([p["mus"], p["sat"]], (1, 1), h, src, dst, ea)
    mus = om.mean(axis=1) + p["mus"]["bias"][0]
    satn = osat.mean(axis=1) + p["sat"]["bias"][0]

    w = (mask == 0).astype(jnp.float32)
    bi = batch.astype(jnp.int32)
    oh = (bi[None, :] == jnp.arange(_NG, dtype=jnp.int32)[:, None])
    oh = jnp.concatenate(
        [oh.astype(jnp.float32), jnp.zeros((_NG, _NP - _N), jnp.float32)], axis=1)
    sw = jnp.zeros((_NP, 128), jnp.float32)
    sw = sw.at[:_N, 0].set(satn * w).at[:_N, 1].set(w)
    pooled = _pool(oh, sw)
    sat = pooled[:, 0] / jnp.maximum(pooled[:, 1], 1.0)
    return (mus, sat)


# speedup vs baseline: 69.3935x; 1.5085x over previous
import functools

import jax
import jax.numpy as jnp
from jax import lax
from jax.experimental import pallas as pl
from jax.experimental.pallas import tpu as pltpu
from jax.experimental.pallas import tpu_sc as plsc

_N = 10000
_E = 640000
_HEADS = 16
_NG = 64
_B = 128
_NW = 32
_CHUNKS = 160
_MAIN_CH = _E // _B
_TAILN = (_NW * _CHUNKS - _MAIN_CH) * _B
_RPT = 632
_ACC_ROWS = _RPT * 16
_TAB = 10008
_ZR = 158
_NP = 10112


def _make_edge_kernel(gs):
    num_sub = len(gs)
    offs, o = [], 0
    for g in gs:
        offs.append(o)
        o += g * 16
    sum_c = o
    w_cols = sum_c + 16 * num_sub
    mesh = plsc.VectorSubcoreMesh(core_axis_name="c", subcore_axis_name="s")

    @functools.partial(
        pl.kernel,
        mesh=mesh,
        compiler_params=pltpu.CompilerParams(use_tc_tiling_on_sc=False),
        out_type=jax.ShapeDtypeStruct((2 * _ACC_ROWS, w_cols), jnp.float32),
        scratch_types=[
            pltpu.VMEM((2, _B), jnp.int32),
            pltpu.VMEM((2, _B), jnp.int32),
            pltpu.VMEM((2, _B, sum_c), jnp.float32),
            pltpu.VMEM((2, _B, sum_c), jnp.float32),
            pltpu.VMEM((2, _B, sum_c), jnp.float32),
            pltpu.VMEM((_B, w_cols), jnp.float32),
            pltpu.VMEM((sum_c,), jnp.float32),
            pltpu.VMEM((_ZR, w_cols), jnp.float32),
            pltpu.SemaphoreType.DMA,
            pltpu.SemaphoreType.DMA,
            pltpu.SemaphoreType.DMA,
            pltpu.SemaphoreType.DMA,
            pltpu.VMEM_SHARED((_ACC_ROWS, w_cols), jnp.float32),
        ],
    )
    def edge_kernel(xlt, xrt, eih, emh, tih, eth, atth, outh,
                    srcb, dstb, xlb, xrb, eb, co_v, att_v, zb_v,
                    semi0, semi1, semr0, semr1, acc):
        c = lax.axis_index("c")
        s = lax.axis_index("s")
        wid = s * 2 + c
        semi = (semi0, semi1)
        semr = (semr0, semr1)
        zero16 = jnp.zeros((16,), jnp.float32)

        def zrow(i, carry):
            for t in range(w_cols // 16):
                zb_v[i, pl.ds(t * 16, 16)] = zero16
            return carry
        lax.fori_loop(0, _ZR, zrow, 0)
        r0 = s * _RPT
        for q in range(_RPT // _ZR):
            pltpu.sync_copy(zb_v, acc.at[pl.ds(r0 + q * _ZR, _ZR)])
        plsc.subcore_barrier()

        pltpu.sync_copy(atth, att_v)
        avecs = [att_v[pl.ds(t * 16, 16)] for t in range(sum_c // 16)]

        def gcof(jj):
            return wid * _CHUNKS + jnp.minimum(jj, _CHUNKS - 1)

        def fire_idx(slot, gc):
            mb = gc * _B
            tb = (gc - _MAIN_CH) * _B

            @pl.when(gc < _MAIN_CH)
            def _():
                pltpu.async_copy(eih.at[0, pl.ds(mb, _B)],
                                 srcb.at[slot], semi[slot])
                pltpu.async_copy(eih.at[1, pl.ds(mb, _B)],
                                 dstb.at[slot], semi[slot])

            @pl.when(gc >= _MAIN_CH)
            def _():
                pltpu.async_copy(tih.at[pl.ds(tb, _B)],
                                 srcb.at[slot], semi[slot])
                pltpu.async_copy(tih.at[pl.ds(tb, _B)],
                                 dstb.at[slot], semi[slot])

        def wait_idx(slot):
            pltpu.make_async_copy(tih.at[pl.ds(0, _B)],
                                  srcb.at[slot], semi[slot]).wait()
            pltpu.make_async_copy(tih.at[pl.ds(0, _B)],
                                  dstb.at[slot], semi[slot]).wait()

        def fire_rows(slot, gc):
            mb = gc * _B
            pltpu.async_copy(xlt.at[srcb.at[slot]], xlb.at[slot], semr[slot])
            pltpu.async_copy(xrt.at[dstb.at[slot]], xrb.at[slot], semr[slot])

            @pl.when(gc < _MAIN_CH)
            def _():
                pltpu.async_copy(emh.at[pl.ds(mb, _B)], eb.at[slot], semr[slot])

            @pl.when(gc >= _MAIN_CH)
            def _():
                pltpu.async_copy(eth.at[pl.ds(0, _B)], eb.at[slot], semr[slot])

        def wait_rows(slot):
            pltpu.make_async_copy(xlt.at[pl.ds(0, _B)],
                                  xlb.at[slot], semr[slot]).wait()
            pltpu.make_async_copy(xrt.at[pl.ds(0, _B)],
                                  xrb.at[slot], semr[slot]).wait()
            pltpu.make_async_copy(emh.at[pl.ds(0, _B)],
                                  eb.at[slot], semr[slot]).wait()

        def compute(slot):
            def quad(q, icarry):
                for r in range(4):
                    i = q * 4 + r
                    for t in range(num_sub):
                        alpha = None
                        xls = []
                        for gi in range(gs[t]):
                            ti = offs[t] // 16 + gi
                            col = offs[t] + gi * 16
                            xlg = xlb[slot, i, pl.ds(col, 16)]
                            xrg = xrb[slot, i, pl.ds(col, 16)]
                            ev = eb[slot, i, pl.ds(col, 16)]
                            m = xlg + xrg + ev
                            m = jnp.maximum(m, m * 0.2)
                            ma = m * avecs[ti]
                            alpha = ma if alpha is None else alpha + ma
                            xls.append(xlg)
                        ex = jnp.exp(alpha)
                        co_v[i, pl.ds(sum_c + t * 16, 16)] = ex
                        for gi in range(gs[t]):
                            col = offs[t] + gi * 16
                            co_v[i, pl.ds(col, 16)] = ex * xls[gi]
                return icarry
            lax.fori_loop(0, _B // 4, quad, 0)

        def half(j, cur, nxt):
            wait_idx(nxt)
            fire_rows(nxt, gcof(j + 1))
            wait_rows(cur)
            compute(cur)
            pltpu.sync_copy(co_v, acc.at[dstb.at[cur]], add=True)
            fire_idx(cur, gcof(j + 2))

        fire_idx(0, gcof(0))
        wait_idx(0)
        fire_rows(0, gcof(0))
        fire_idx(1, gcof(1))

        def pair(p, carry):
            half(p * 2, 0, 1)
            half(p * 2 + 1, 1, 0)
            return carry
        lax.fori_loop(0, _CHUNKS // 2, pair, 0)

        wait_idx(1)
        wait_rows(0)

        plsc.subcore_barrier()
        for q in range(_RPT // _ZR):
            pltpu.sync_copy(acc.at[pl.ds(r0 + q * _ZR, _ZR)], zb_v)
            pltpu.sync_copy(zb_v, outh.at[pl.ds(c * _ACC_ROWS + r0 + q * _ZR,
                                                _ZR)])

    return edge_kernel


_edge_k4 = _make_edge_kernel((4,))
_edge_k11 = _make_edge_kernel((1, 1))


def _pool_body(lhs_ref, oh_ref, o_ref):
    o_ref[...] = jnp.dot(lhs_ref[...], oh_ref[...],
                         preferred_element_type=jnp.float32,
                         precision=jax.lax.Precision.HIGHEST)


def _pool(lhs, oh):
    return pl.pallas_call(
        _pool_body,
        out_shape=jax.ShapeDtypeStruct((8, 128), jnp.float32),
    )(lhs, oh)


def _to_t(a, g):
    return a.reshape(-1, _HEADS, g).transpose(0, 2, 1).reshape(-1, g * _HEADS)


def _edge_pass(plist, gs, h, ei, ea, ea_mean, tidx):
    xls, xrs, wes, atts = [], [], [], []
    for p, g in zip(plist, gs):
        cc = g * 16
        xls.append(_to_t(h @ p["Wl"] + p["bl"], g))
        xrs.append(_to_t(h @ p["Wr"] + p["br"], g))
        wes.append(p["We"].reshape(4, _HEADS, g).transpose(0, 2, 1).reshape(4, cc))
        atts.append(p["att"].transpose(1, 0).reshape(cc))
    sum_c = sum(g * 16 for g in gs)
    pad = jnp.zeros((_TAB - _N, sum_c), jnp.float32)
    xlt = jnp.concatenate([jnp.concatenate(xls, axis=1), pad], axis=0)
    xrt = jnp.concatenate([jnp.concatenate(xrs, axis=1), pad], axis=0)
    wet = jnp.concatenate(wes, axis=1)
    att = jnp.concatenate(atts)
    e_main = ea @ wet
    e_tail = jnp.broadcast_to(ea_mean @ wet, (_B, sum_c))
    ek = _edge_k4 if len(gs) == 1 else _edge_k11
    parts = ek(xlt, xrt, ei, e_main, tidx, e_tail, att)
    acc = parts.reshape(2, _ACC_ROWS, -1).sum(0)[:_N]
    outs = []
    for t, g in enumerate(gs):
        cc = g * 16
        off = sum(gg * 16 for gg in gs[:t])
        num = acc[:, off:off + cc]
        den = acc[:, sum_c + t * 16: sum_c + (t + 1) * 16]
        out_t = num / (jnp.tile(den, (1, g)) + 1e-16)
        outs.append(out_t.reshape(_N, g, _HEADS).transpose(0, 2, 1)
                    .reshape(_N, _HEADS * g))
    return outs


def _bn(h, g, b):
    mu = h.mean(0)
    var = h.var(0)
    return (h - mu) / jnp.sqrt(var + 1e-5) * g + b


def kernel(x, edge_index, edge_attr, mask, batch, params):
    ei = edge_index.astype(jnp.int32)
    tidx = jnp.concatenate([
        jnp.arange(_N, dtype=jnp.int32),
        jnp.full((_TAILN - _N,), _N, jnp.int32),
    ])
    ea_mean = edge_attr.mean(0)
    p = params

    args = (ei, edge_attr, ea_mean, tidx)
    h = _edge_pass([p["init"]], (4,), x, *args)[0] + p["init"]["bias"]
    h = jax.nn.leaky_relu(_bn(h, p["bn0_g"], p["bn0_b"]), 0.01)
    h = _edge_pass([p["hid"]], (4,), h, *args)[0] + p["hid"]["bias"]
    h = jax.nn.leaky_relu(_bn(h, p["bn1_g"], p["bn1_b"]), 0.01)
    om, osat = _edge_pass([p["mus"], p["sat"]], (1, 1), h, *args)
    mus = om.mean(axis=1) + p["mus"]["bias"][0]
    satn = osat.mean(axis=1) + p["sat"]["bias"][0]

    w = (mask == 0).astype(jnp.float32)
    batch_p = jnp.pad(batch.astype(jnp.int32), (0, _NP - _N),
                      constant_values=_NG)
    oh = (batch_p[:, None] == jnp.arange(128, dtype=jnp.int32)[None, :]
          ).astype(jnp.float32)
    lhs = jnp.pad(jnp.stack([satn * w, w]), ((0, 6), (0, _NP - _N)))
    pooled = _pool(lhs, oh)
    sat = pooled[0, :_NG] / jnp.maximum(pooled[1, :_NG], 1.0)
    return (mus, sat)


# speedup vs baseline: 94.8473x; 1.3668x over previous
import functools

import jax
import jax.numpy as jnp
from jax import lax
from jax.experimental import pallas as pl
from jax.experimental.pallas import tpu as pltpu
from jax.experimental.pallas import tpu_sc as plsc

_N = 10000
_E = 640000
_HEADS = 16
_NG = 64
_B = 128
_NW = 32
_CHUNKS = 160
_MAIN_CH = _E // _B
_TAILN = (_NW * _CHUNKS - _MAIN_CH) * _B
_RPT = 632
_ACC_ROWS = _RPT * 16
_TAB = 10008
_ZR = 158
_NP = 10112


def _make_edge_kernel(gs):
    num_sub = len(gs)
    offs, o = [], 0
    for g in gs:
        offs.append(o)
        o += g * 16
    sum_c = o
    w_cols = sum_c + 16 * num_sub
    mesh = plsc.VectorSubcoreMesh(core_axis_name="c", subcore_axis_name="s")

    @functools.partial(
        pl.kernel,
        mesh=mesh,
        compiler_params=pltpu.CompilerParams(use_tc_tiling_on_sc=False),
        out_type=jax.ShapeDtypeStruct((2 * _ACC_ROWS, w_cols), jnp.float32),
        scratch_types=[
            pltpu.VMEM((2, _B), jnp.int32),
            pltpu.VMEM((2, _B), jnp.int32),
            pltpu.VMEM((2, _B, sum_c), jnp.float32),
            pltpu.VMEM((2, _B, sum_c), jnp.float32),
            pltpu.VMEM((2, _B, sum_c), jnp.float32),
            pltpu.VMEM((_B, w_cols), jnp.float32),
            pltpu.VMEM((sum_c,), jnp.float32),
            pltpu.VMEM((_ZR, w_cols), jnp.float32),
            pltpu.SemaphoreType.DMA,
            pltpu.SemaphoreType.DMA,
            pltpu.SemaphoreType.DMA,
            pltpu.SemaphoreType.DMA,
            pltpu.VMEM_SHARED((_ACC_ROWS, w_cols), jnp.float32),
        ],
    )
    def edge_kernel(xlt, xrt, eih, emh, tih, eth, atth, outh,
                    srcb, dstb, xlb, xrb, eb, co_v, att_v, zb_v,
                    semi0, semi1, semr0, semr1, acc):
        c = lax.axis_index("c")
        s = lax.axis_index("s")
        wid = s * 2 + c
        semi = (semi0, semi1)
        semr = (semr0, semr1)
        zero16 = jnp.zeros((16,), jnp.float32)

        def zrow(i, carry):
            for t in range(w_cols // 16):
                zb_v[i, pl.ds(t * 16, 16)] = zero16
            return carry
        lax.fori_loop(0, _ZR, zrow, 0)
        r0 = s * _RPT
        for q in range(_RPT // _ZR):
            pltpu.sync_copy(zb_v, acc.at[pl.ds(r0 + q * _ZR, _ZR)])
        plsc.subcore_barrier()

        pltpu.sync_copy(atth, att_v)
        avecs = [att_v[pl.ds(t * 16, 16)] for t in range(sum_c // 16)]

        def gcof(jj):
            return wid * _CHUNKS + jnp.minimum(jj, _CHUNKS - 1)

        def fire_idx(slot, gc):
            mb = gc * _B
            tb = (gc - _MAIN_CH) * _B

            @pl.when(gc < _MAIN_CH)
            def _():
                pltpu.async_copy(eih.at[0, pl.ds(mb, _B)],
                                 srcb.at[slot], semi[slot])
                pltpu.async_copy(eih.at[1, pl.ds(mb, _B)],
                                 dstb.at[slot], semi[slot])

            @pl.when(gc >= _MAIN_CH)
            def _():
                pltpu.async_copy(tih.at[pl.ds(tb, _B)],
                                 srcb.at[slot], semi[slot])
                pltpu.async_copy(tih.at[pl.ds(tb, _B)],
                                 dstb.at[slot], semi[slot])

        def wait_idx(slot):
            pltpu.make_async_copy(tih.at[pl.ds(0, _B)],
                                  srcb.at[slot], semi[slot]).wait()
            pltpu.make_async_copy(tih.at[pl.ds(0, _B)],
                                  dstb.at[slot], semi[slot]).wait()

        def fire_rows(slot, gc):
            mb = gc * _B
            pltpu.async_copy(xlt.at[srcb.at[slot]], xlb.at[slot], semr[slot])
            pltpu.async_copy(xrt.at[dstb.at[slot]], xrb.at[slot], semr[slot])

            @pl.when(gc < _MAIN_CH)
            def _():
                pltpu.async_copy(emh.at[pl.ds(mb, _B)], eb.at[slot], semr[slot])

            @pl.when(gc >= _MAIN_CH)
            def _():
                pltpu.async_copy(eth.at[pl.ds(0, _B)], eb.at[slot], semr[slot])

        def wait_rows(slot):
            pltpu.make_async_copy(xlt.at[pl.ds(0, _B)],
                                  xlb.at[slot], semr[slot]).wait()
            pltpu.make_async_copy(xrt.at[pl.ds(0, _B)],
                                  xrb.at[slot], semr[slot]).wait()
            pltpu.make_async_copy(emh.at[pl.ds(0, _B)],
                                  eb.at[slot], semr[slot]).wait()

        def compute(slot):
            def quad(q, icarry):
                alphas, kept = [], []
                for r in range(4):
                    i = q * 4 + r
                    for t in range(num_sub):
                        alpha = None
                        xls = []
                        for gi in range(gs[t]):
                            ti = offs[t] // 16 + gi
                            col = offs[t] + gi * 16
                            xlg = xlb[slot, i, pl.ds(col, 16)]
                            xrg = xrb[slot, i, pl.ds(col, 16)]
                            ev = eb[slot, i, pl.ds(col, 16)]
                            m = xlg + xrg + ev
                            m = jnp.maximum(m, m * 0.2)
                            ma = m * avecs[ti]
                            alpha = ma if alpha is None else alpha + ma
                            xls.append(xlg)
                        alphas.append(alpha)
                        kept.append(xls)
                exs = [jnp.exp(a) for a in alphas]
                k = 0
                for r in range(4):
                    i = q * 4 + r
                    for t in range(num_sub):
                        ex = exs[k]
                        co_v[i, pl.ds(sum_c + t * 16, 16)] = ex
                        for gi in range(gs[t]):
                            col = offs[t] + gi * 16
                            co_v[i, pl.ds(col, 16)] = ex * kept[k][gi]
                        k += 1
                return icarry
            lax.fori_loop(0, _B // 4, quad, 0, unroll=2)

        def half(j, cur, nxt):
            wait_idx(nxt)
            fire_rows(nxt, gcof(j + 1))
            wait_rows(cur)
            compute(cur)
            pltpu.sync_copy(co_v, acc.at[dstb.at[cur]], add=True)
            fire_idx(cur, gcof(j + 2))

        fire_idx(0, gcof(0))
        wait_idx(0)
        fire_rows(0, gcof(0))
        fire_idx(1, gcof(1))

        def pair(p, carry):
            half(p * 2, 0, 1)
            half(p * 2 + 1, 1, 0)
            return carry
        lax.fori_loop(0, _CHUNKS // 2, pair, 0)

        wait_idx(1)
        wait_rows(0)

        plsc.subcore_barrier()
        for q in range(_RPT // _ZR):
            pltpu.sync_copy(acc.at[pl.ds(r0 + q * _ZR, _ZR)], zb_v)
            pltpu.sync_copy(zb_v, outh.at[pl.ds(c * _ACC_ROWS + r0 + q * _ZR,
                                                _ZR)])

    return edge_kernel


_edge_k4 = _make_edge_kernel((4,))
_edge_k11 = _make_edge_kernel((1, 1))


def _pool_body(lhs_ref, oh_ref, o_ref):
    o_ref[...] = jnp.dot(lhs_ref[...], oh_ref[...],
                         preferred_element_type=jnp.float32,
                         precision=jax.lax.Precision.HIGHEST)


def _pool(lhs, oh):
    return pl.pallas_call(
        _pool_body,
        out_shape=jax.ShapeDtypeStruct((8, 128), jnp.float32),
    )(lhs, oh)


def _to_t(a, g):
    return a.reshape(-1, _HEADS, g).transpose(0, 2, 1).reshape(-1, g * _HEADS)


def _edge_pass(plist, gs, h, ei, ea, ea_mean, tidx):
    xls, xrs, wes, atts = [], [], [], []
    for p, g in zip(plist, gs):
        cc = g * 16
        xls.append(_to_t(h @ p["Wl"] + p["bl"], g))
        xrs.append(_to_t(h @ p["Wr"] + p["br"], g))
        wes.append(p["We"].reshape(4, _HEADS, g).transpose(0, 2, 1).reshape(4, cc))
        atts.append(p["att"].transpose(1, 0).reshape(cc))
    sum_c = sum(g * 16 for g in gs)
    pad = jnp.zeros((_TAB - _N, sum_c), jnp.float32)
    xlt = jnp.concatenate([jnp.concatenate(xls, axis=1), pad], axis=0)
    xrt = jnp.concatenate([jnp.concatenate(xrs, axis=1), pad], axis=0)
    wet = jnp.concatenate(wes, axis=1)
    att = jnp.concatenate(atts)
    e_main = ea @ wet
    e_tail = jnp.broadcast_to(ea_mean @ wet, (_B, sum_c))
    ek = _edge_k4 if len(gs) == 1 else _edge_k11
    parts = ek(xlt, xrt, ei, e_main, tidx, e_tail, att)
    acc = parts.reshape(2, _ACC_ROWS, -1).sum(0)[:_N]
    outs = []
    for t, g in enumerate(gs):
        cc = g * 16
        off = sum(gg * 16 for gg in gs[:t])
        num = acc[:, off:off + cc]
        den = acc[:, sum_c + t * 16: sum_c + (t + 1) * 16]
        out_t = num / (jnp.tile(den, (1, g)) + 1e-16)
        outs.append(out_t.reshape(_N, g, _HEADS).transpose(0, 2, 1)
                    .reshape(_N, _HEADS * g))
    return outs


def _bn(h, g, b):
    mu = h.mean(0)
    var = h.var(0)
    return (h - mu) / jnp.sqrt(var + 1e-5) * g + b


def kernel(x, edge_index, edge_attr, mask, batch, params):
    ei = edge_index.astype(jnp.int32)
    tidx = jnp.concatenate([
        jnp.arange(_N, dtype=jnp.int32),
        jnp.full((_TAILN - _N,), _N, jnp.int32),
    ])
    ea_mean = edge_attr.mean(0)
    p = params

    args = (ei, edge_attr, ea_mean, tidx)
    h = _edge_pass([p["init"]], (4,), x, *args)[0] + p["init"]["bias"]
    h = jax.nn.leaky_relu(_bn(h, p["bn0_g"], p["bn0_b"]), 0.01)
    h = _edge_pass([p["hid"]], (4,), h, *args)[0] + p["hid"]["bias"]
    h = jax.nn.leaky_relu(_bn(h, p["bn1_g"], p["bn1_b"]), 0.01)
    om, osat = _edge_pass([p["mus"], p["sat"]], (1, 1), h, *args)
    mus = om.mean(axis=1) + p["mus"]["bias"][0]
    satn = osat.mean(axis=1) + p["sat"]["bias"][0]

    w = (mask == 0).astype(jnp.float32)
    batch_p = jnp.pad(batch.astype(jnp.int32), (0, _NP - _N),
                      constant_values=_NG)
    oh = (batch_p[:, None] == jnp.arange(128, dtype=jnp.int32)[None, :]
          ).astype(jnp.float32)
    lhs = jnp.pad(jnp.stack([satn * w, w]), ((0, 6), (0, _NP - _N)))
    pooled = _pool(lhs, oh)
    sat = pooled[0, :_NG] / jnp.maximum(pooled[1, :_NG], 1.0)
    return (mus, sat)


# speedup vs baseline: 99.4536x; 1.0486x over previous
import functools

import jax
import jax.numpy as jnp
from jax import lax
from jax.experimental import pallas as pl
from jax.experimental.pallas import tpu as pltpu
from jax.experimental.pallas import tpu_sc as plsc

_N = 10000
_E = 640000
_HEADS = 16
_NG = 64
_B = 128
_NW = 32
_CHUNKS = 160
_MAIN_CH = _E // _B
_TAILN = (_NW * _CHUNKS - _MAIN_CH) * _B
_RPT = 632
_ACC_ROWS = _RPT * 16
_TAB = 10008
_ZR = 79
_NP = 10112


def _make_edge_kernel(gs):
    num_sub = len(gs)
    offs, o = [], 0
    for g in gs:
        offs.append(o)
        o += g * 16
    sum_c = o
    w_cols = sum_c + 16 * num_sub
    mesh = plsc.VectorSubcoreMesh(core_axis_name="c", subcore_axis_name="s")

    @functools.partial(
        pl.kernel,
        mesh=mesh,
        compiler_params=pltpu.CompilerParams(use_tc_tiling_on_sc=False),
        out_type=jax.ShapeDtypeStruct((2 * _ACC_ROWS, w_cols), jnp.float32),
        scratch_types=[
            pltpu.VMEM((2, _B), jnp.int32),
            pltpu.VMEM((2, _B), jnp.int32),
            pltpu.VMEM((2, _B, sum_c), jnp.float32),
            pltpu.VMEM((2, _B, sum_c), jnp.float32),
            pltpu.VMEM((2, _B, sum_c), jnp.float32),
            pltpu.VMEM((2, _B, w_cols), jnp.float32),
            pltpu.VMEM((2, _B), jnp.int32),
            pltpu.VMEM((sum_c,), jnp.float32),
            pltpu.VMEM((_ZR, w_cols), jnp.float32),
            pltpu.SemaphoreType.DMA,
            pltpu.SemaphoreType.DMA,
            pltpu.SemaphoreType.DMA,
            pltpu.SemaphoreType.DMA,
            pltpu.SemaphoreType.DMA,
            pltpu.SemaphoreType.DMA,
            pltpu.VMEM_SHARED((_ACC_ROWS, w_cols), jnp.float32),
        ],
    )
    def edge_kernel(xlt, xrt, eih, emh, tih, eth, atth, outh,
                    srcb, dstb, xlb, xrb, eb, co_v, dsc, att_v, zb_v,
                    semi0, semi1, semr0, semr1, sems0, sems1, acc):
        c = lax.axis_index("c")
        s = lax.axis_index("s")
        wid = s * 2 + c
        semi = (semi0, semi1)
        semr = (semr0, semr1)
        sems = (sems0, sems1)
        zero16 = jnp.zeros((16,), jnp.float32)

        def zrow(i, carry):
            for t in range(w_cols // 16):
                zb_v[i, pl.ds(t * 16, 16)] = zero16
            return carry
        lax.fori_loop(0, _ZR, zrow, 0)
        r0 = s * _RPT
        for q in range(_RPT // _ZR):
            pltpu.sync_copy(zb_v, acc.at[pl.ds(r0 + q * _ZR, _ZR)])
        plsc.subcore_barrier()

        pltpu.sync_copy(atth, att_v)
        avecs = [att_v[pl.ds(t * 16, 16)] for t in range(sum_c // 16)]

        def gcof(jj):
            return wid * _CHUNKS + jnp.minimum(jj, _CHUNKS - 1)

        def fire_idx(slot, gc):
            mb = gc * _B
            tb = (gc - _MAIN_CH) * _B

            @pl.when(gc < _MAIN_CH)
            def _():
                pltpu.async_copy(eih.at[0, pl.ds(mb, _B)],
                                 srcb.at[slot], semi[slot])
                pltpu.async_copy(eih.at[1, pl.ds(mb, _B)],
                                 dstb.at[slot], semi[slot])

            @pl.when(gc >= _MAIN_CH)
            def _():
                pltpu.async_copy(tih.at[pl.ds(tb, _B)],
                                 srcb.at[slot], semi[slot])
                pltpu.async_copy(tih.at[pl.ds(tb, _B)],
                                 dstb.at[slot], semi[slot])

        def wait_idx(slot):
            pltpu.make_async_copy(tih.at[pl.ds(0, _B)],
                                  srcb.at[slot], semi[slot]).wait()
            pltpu.make_async_copy(tih.at[pl.ds(0, _B)],
                                  dstb.at[slot], semi[slot]).wait()

        def fire_rows(slot, gc):
            mb = gc * _B
            pltpu.async_copy(xlt.at[srcb.at[slot]], xlb.at[slot], semr[slot])
            pltpu.async_copy(xrt.at[dstb.at[slot]], xrb.at[slot], semr[slot])

            @pl.when(gc < _MAIN_CH)
            def _():
                pltpu.async_copy(emh.at[pl.ds(mb, _B)], eb.at[slot], semr[slot])

            @pl.when(gc >= _MAIN_CH)
            def _():
                pltpu.async_copy(eth.at[pl.ds(0, _B)], eb.at[slot], semr[slot])

        def wait_rows(slot):
            pltpu.make_async_copy(xlt.at[pl.ds(0, _B)],
                                  xlb.at[slot], semr[slot]).wait()
            pltpu.make_async_copy(xrt.at[pl.ds(0, _B)],
                                  xrb.at[slot], semr[slot]).wait()
            pltpu.make_async_copy(emh.at[pl.ds(0, _B)],
                                  eb.at[slot], semr[slot]).wait()

        def wait_scatter(slot):
            pltpu.make_async_copy(co_v.at[slot], acc.at[pl.ds(0, _B)],
                                  sems[slot]).wait()

        def compute(slot):
            def quad(q, icarry):
                alphas, kept = [], []
                for r in range(4):
                    i = q * 4 + r
                    for t in range(num_sub):
                        alpha = None
                        xls = []
                        for gi in range(gs[t]):
                            ti = offs[t] // 16 + gi
                            col = offs[t] + gi * 16
                            xlg = xlb[slot, i, pl.ds(col, 16)]
                            xrg = xrb[slot, i, pl.ds(col, 16)]
                            ev = eb[slot, i, pl.ds(col, 16)]
                            m = xlg + xrg + ev
                            m = jnp.maximum(m, m * 0.2)
                            ma = m * avecs[ti]
                            alpha = ma if alpha is None else alpha + ma
                            xls.append(xlg)
                        alphas.append(alpha)
                        kept.append(xls)
                exs = [jnp.exp(a) for a in alphas]
                k = 0
                for r in range(4):
                    i = q * 4 + r
                    for t in range(num_sub):
                        ex = exs[k]
                        co_v[slot, i, pl.ds(sum_c + t * 16, 16)] = ex
                        for gi in range(gs[t]):
                            col = offs[t] + gi * 16
                            co_v[slot, i, pl.ds(col, 16)] = ex * kept[k][gi]
                        k += 1
                return icarry
            lax.fori_loop(0, _B // 4, quad, 0, unroll=2)

        def half(j, cur, nxt):
            wait_idx(nxt)
            fire_rows(nxt, gcof(j + 1))
            wait_rows(cur)

            @pl.when(j >= 2)
            def _():
                wait_scatter(cur)
            compute(cur)
            for t in range(_B // 16):
                dsc[cur, pl.ds(t * 16, 16)] = dstb[cur, pl.ds(t * 16, 16)]
            pltpu.async_copy(co_v.at[cur], acc.at[dsc.at[cur]], sems[cur],
                             add=True)
            fire_idx(cur, gcof(j + 2))

        fire_idx(0, gcof(0))
        wait_idx(0)
        fire_rows(0, gcof(0))
        fire_idx(1, gcof(1))

        def pair(p, carry):
            half(p * 2, 0, 1)
            half(p * 2 + 1, 1, 0)
            return carry
        lax.fori_loop(0, _CHUNKS // 2, pair, 0)

        wait_idx(1)
        wait_rows(0)
        wait_scatter(0)
        wait_scatter(1)

        plsc.subcore_barrier()
        for q in range(_RPT // _ZR):
            pltpu.sync_copy(acc.at[pl.ds(r0 + q * _ZR, _ZR)], zb_v)
            pltpu.sync_copy(zb_v, outh.at[pl.ds(c * _ACC_ROWS + r0 + q * _ZR,
                                                _ZR)])

    return edge_kernel


_edge_k4 = _make_edge_kernel((4,))
_edge_k11 = _make_edge_kernel((1, 1))


def _pool_body(lhs_ref, oh_ref, o_ref):
    o_ref[...] = jnp.dot(lhs_ref[...], oh_ref[...],
                         preferred_element_type=jnp.float32,
                         precision=jax.lax.Precision.HIGHEST)


def _pool(lhs, oh):
    return pl.pallas_call(
        _pool_body,
        out_shape=jax.ShapeDtypeStruct((8, 128), jnp.float32),
    )(lhs, oh)


def _to_t(a, g):
    return a.reshape(-1, _HEADS, g).transpose(0, 2, 1).reshape(-1, g * _HEADS)


def _edge_pass(plist, gs, h, ei, ea, ea_mean, tidx):
    xls, xrs, wes, atts = [], [], [], []
    for p, g in zip(plist, gs):
        cc = g * 16
        xls.append(_to_t(h @ p["Wl"] + p["bl"], g))
        xrs.append(_to_t(h @ p["Wr"] + p["br"], g))
        wes.append(p["We"].reshape(4, _HEADS, g).transpose(0, 2, 1).reshape(4, cc))
        atts.append(p["att"].transpose(1, 0).reshape(cc))
    sum_c = sum(g * 16 for g in gs)
    pad = jnp.zeros((_TAB - _N, sum_c), jnp.float32)
    xlt = jnp.concatenate([jnp.concatenate(xls, axis=1), pad], axis=0)
    xrt = jnp.concatenate([jnp.concatenate(xrs, axis=1), pad], axis=0)
    wet = jnp.concatenate(wes, axis=1)
    att = jnp.concatenate(atts)
    e_main = ea @ wet
    e_tail = jnp.broadcast_to(ea_mean @ wet, (_B, sum_c))
    ek = _edge_k4 if len(gs) == 1 else _edge_k11
    parts = ek(xlt, xrt, ei, e_main, tidx, e_tail, att)
    acc = parts.reshape(2, _ACC_ROWS, -1).sum(0)[:_N]
    outs = []
    for t, g in enumerate(gs):
        cc = g * 16
        off = sum(gg * 16 for gg in gs[:t])
        num = acc[:, off:off + cc]
        den = acc[:, sum_c + t * 16: sum_c + (t + 1) * 16]
        out_t = num / (jnp.tile(den, (1, g)) + 1e-16)
        outs.append(out_t.reshape(_N, g, _HEADS).transpose(0, 2, 1)
                    .reshape(_N, _HEADS * g))
    return outs


def _bn(h, g, b):
    mu = h.mean(0)
    var = h.var(0)
    return (h - mu) / jnp.sqrt(var + 1e-5) * g + b


def kernel(x, edge_index, edge_attr, mask, batch, params):
    ei = edge_index.astype(jnp.int32)
    tidx = jnp.concatenate([
        jnp.arange(_N, dtype=jnp.int32),
        jnp.full((_TAILN - _N,), _N, jnp.int32),
    ])
    ea_mean = edge_attr.mean(0)
    p = params

    args = (ei, edge_attr, ea_mean, tidx)
    h = _edge_pass([p["init"]], (4,), x, *args)[0] + p["init"]["bias"]
    h = jax.nn.leaky_relu(_bn(h, p["bn0_g"], p["bn0_b"]), 0.01)
    h = _edge_pass([p["hid"]], (4,), h, *args)[0] + p["hid"]["bias"]
    h = jax.nn.leaky_relu(_bn(h, p["bn1_g"], p["bn1_b"]), 0.01)
    om, osat = _edge_pass([p["mus"], p["sat"]], (1, 1), h, *args)
    mus = om.mean(axis=1) + p["mus"]["bias"][0]
    satn = osat.mean(axis=1) + p["sat"]["bias"][0]

    w = (mask == 0).astype(jnp.float32)
    batch_p = jnp.pad(batch.astype(jnp.int32), (0, _NP - _N),
                      constant_values=_NG)
    oh = (batch_p[:, None] == jnp.arange(128, dtype=jnp.int32)[None, :]
          ).astype(jnp.float32)
    lhs = jnp.pad(jnp.stack([satn * w, w]), ((0, 6), (0, _NP - _N)))
    pooled = _pool(lhs, oh)
    sat = pooled[0, :_NG] / jnp.maximum(pooled[1, :_NG], 1.0)
    return (mus, sat)


# speedup vs baseline: 101.8251x; 1.0238x over previous
import functools

import jax
import jax.numpy as jnp
from jax import lax
from jax.experimental import pallas as pl
from jax.experimental.pallas import tpu as pltpu
from jax.experimental.pallas import tpu_sc as plsc

_N = 10000
_E = 640000
_HEADS = 16
_NG = 64
_B = 128
_NW = 32
_CHUNKS = 160
_MAIN_CH = _E // _B
_TAILN = (_NW * _CHUNKS - _MAIN_CH) * _B
_RPT = 632
_ACC_ROWS = _RPT * 16
_TAB = 10008
_ZR = 79
_NP = 10112


def _make_edge_kernel(gs):
    num_sub = len(gs)
    offs, o = [], 0
    for g in gs:
        offs.append(o)
        o += g * 16
    sum_c = o
    w_cols = sum_c + 16 * num_sub
    mesh = plsc.VectorSubcoreMesh(core_axis_name="c", subcore_axis_name="s")

    @functools.partial(
        pl.kernel,
        mesh=mesh,
        compiler_params=pltpu.CompilerParams(use_tc_tiling_on_sc=False),
        out_type=jax.ShapeDtypeStruct((2 * _ACC_ROWS, w_cols), jnp.float32),
        scratch_types=[
            pltpu.VMEM((2, _B), jnp.int32),
            pltpu.VMEM((2, _B), jnp.int32),
            pltpu.VMEM((2, _B, sum_c), jnp.float32),
            pltpu.VMEM((2, _B, sum_c), jnp.float32),
            pltpu.VMEM((2, _B, sum_c), jnp.float32),
            pltpu.VMEM((2, _B, w_cols), jnp.float32),
            pltpu.VMEM((2, _B), jnp.int32),
            pltpu.VMEM((sum_c,), jnp.float32),
            pltpu.VMEM((_ZR, w_cols), jnp.float32),
            pltpu.SemaphoreType.DMA,
            pltpu.SemaphoreType.DMA,
            pltpu.SemaphoreType.DMA,
            pltpu.SemaphoreType.DMA,
            pltpu.SemaphoreType.DMA,
            pltpu.SemaphoreType.DMA,
            pltpu.VMEM_SHARED((_ACC_ROWS, w_cols), jnp.float32),
        ],
    )
    def edge_kernel(xlt, xrt, eih, emh, tih, eth, atth, outh,
                    srcb, dstb, xlb, xrb, eb, co_v, dsc, att_v, zb_v,
                    semi0, semi1, semr0, semr1, sems0, sems1, acc):
        c = lax.axis_index("c")
        s = lax.axis_index("s")
        wid = s * 2 + c
        semi = (semi0, semi1)
        semr = (semr0, semr1)
        sems = (sems0, sems1)
        zero16 = jnp.zeros((16,), jnp.float32)

        def zrow(i, carry):
            for t in range(w_cols // 16):
                zb_v[i, pl.ds(t * 16, 16)] = zero16
            return carry
        lax.fori_loop(0, _ZR, zrow, 0)
        r0 = s * _RPT
        for q in range(_RPT // _ZR):
            pltpu.sync_copy(zb_v, acc.at[pl.ds(r0 + q * _ZR, _ZR)])
        plsc.subcore_barrier()

        pltpu.sync_copy(atth, att_v)
        avecs = [att_v[pl.ds(t * 16, 16)] for t in range(sum_c // 16)]

        def gcof(jj):
            return wid * _CHUNKS + jnp.minimum(jj, _CHUNKS - 1)

        def fire_idx(slot, gc):
            mb = gc * _B
            tb = (gc - _MAIN_CH) * _B

            @pl.when(gc < _MAIN_CH)
            def _():
                pltpu.async_copy(eih.at[0, pl.ds(mb, _B)],
                                 srcb.at[slot], semi[slot])
                pltpu.async_copy(eih.at[1, pl.ds(mb, _B)],
                                 dstb.at[slot], semi[slot])

            @pl.when(gc >= _MAIN_CH)
            def _():
                pltpu.async_copy(tih.at[pl.ds(tb, _B)],
                                 srcb.at[slot], semi[slot])
                pltpu.async_copy(tih.at[pl.ds(tb, _B)],
                                 dstb.at[slot], semi[slot])

        def wait_idx(slot):
            pltpu.make_async_copy(tih.at[pl.ds(0, _B)],
                                  srcb.at[slot], semi[slot]).wait()
            pltpu.make_async_copy(tih.at[pl.ds(0, _B)],
                                  dstb.at[slot], semi[slot]).wait()

        def fire_rows(slot, gc):
            mb = gc * _B
            pltpu.async_copy(xlt.at[srcb.at[slot]], xlb.at[slot], semr[slot])
            pltpu.async_copy(xrt.at[dstb.at[slot]], xrb.at[slot], semr[slot])

            @pl.when(gc < _MAIN_CH)
            def _():
                pltpu.async_copy(emh.at[pl.ds(mb, _B)], eb.at[slot], semr[slot])

            @pl.when(gc >= _MAIN_CH)
            def _():
                pltpu.async_copy(eth.at[pl.ds(0, _B)], eb.at[slot], semr[slot])

        def wait_rows(slot):
            pltpu.make_async_copy(xlt.at[pl.ds(0, _B)],
                                  xlb.at[slot], semr[slot]).wait()
            pltpu.make_async_copy(xrt.at[pl.ds(0, _B)],
                                  xrb.at[slot], semr[slot]).wait()
            pltpu.make_async_copy(emh.at[pl.ds(0, _B)],
                                  eb.at[slot], semr[slot]).wait()

        def wait_scatter(slot):
            pltpu.make_async_copy(co_v.at[slot], acc.at[pl.ds(0, _B)],
                                  sems[slot]).wait()

        def compute(slot):
            def quad(q, icarry):
                alphas, kept = [], []
                for r in range(4):
                    i = q * 4 + r
                    for t in range(num_sub):
                        alpha = None
                        xls = []
                        for gi in range(gs[t]):
                            ti = offs[t] // 16 + gi
                            col = offs[t] + gi * 16
                            xlg = xlb[slot, i, pl.ds(col, 16)]
                            xrg = xrb[slot, i, pl.ds(col, 16)]
                            ev = eb[slot, i, pl.ds(col, 16)]
                            m = xlg + xrg + ev
                            m = jnp.maximum(m, m * 0.2)
                            ma = m * avecs[ti]
                            alpha = ma if alpha is None else alpha + ma
                            xls.append(xlg)
                        alphas.append(alpha)
                        kept.append(xls)
                exs = [jnp.exp(a) for a in alphas]
                k = 0
                for r in range(4):
                    i = q * 4 + r
                    for t in range(num_sub):
                        ex = exs[k]
                        co_v[slot, i, pl.ds(sum_c + t * 16, 16)] = ex
                        for gi in range(gs[t]):
                            col = offs[t] + gi * 16
                            co_v[slot, i, pl.ds(col, 16)] = ex * kept[k][gi]
                        k += 1
                return icarry
            lax.fori_loop(0, _B // 4, quad, 0, unroll=2)

        def half(j, cur, nxt):
            wait_idx(nxt)
            fire_rows(nxt, gcof(j + 1))
            wait_rows(cur)

            @pl.when(j >= 2)
            def _():
                wait_scatter(cur)
            compute(cur)
            for t in range(_B // 16):
                dsc[cur, pl.ds(t * 16, 16)] = dstb[cur, pl.ds(t * 16, 16)]
            pltpu.async_copy(co_v.at[cur], acc.at[dsc.at[cur]], sems[cur],
                             add=True)
            fire_idx(cur, gcof(j + 2))

        fire_idx(0, gcof(0))
        wait_idx(0)
        fire_rows(0, gcof(0))
        fire_idx(1, gcof(1))

        def pair(p, carry):
            half(p * 2, 0, 1)
            half(p * 2 + 1, 1, 0)
            return carry
        lax.fori_loop(0, _CHUNKS // 2, pair, 0)

        wait_idx(1)
        wait_rows(0)
        wait_scatter(0)
        wait_scatter(1)

        plsc.subcore_barrier()
        for q in range(_RPT // _ZR):
            pltpu.sync_copy(acc.at[pl.ds(r0 + q * _ZR, _ZR)], zb_v)
            pltpu.sync_copy(zb_v, outh.at[pl.ds(c * _ACC_ROWS + r0 + q * _ZR,
                                                _ZR)])

    return edge_kernel


_edge_k4 = _make_edge_kernel((4,))
_edge_k11 = _make_edge_kernel((1, 1))


def _pool_body(lhs_ref, oh_ref, o_ref):
    o_ref[...] = jnp.dot(lhs_ref[...], oh_ref[...],
                         preferred_element_type=jnp.float32,
                         precision=jax.lax.Precision.HIGHEST)


def _pool(lhs, oh):
    return pl.pallas_call(
        _pool_body,
        out_shape=jax.ShapeDtypeStruct((8, 128), jnp.float32),
    )(lhs, oh)


def _perm_out(g):
    return jnp.array([h * g + oc for oc in range(g) for h in range(_HEADS)])


def _edge_pass(plist, gs, h, ei, ea, ea_mean, tidx, inperm):
    xls, xrs, wes, atts, pos = [], [], [], [], []
    for p, g in zip(plist, gs):
        cc = g * 16
        po = _perm_out(g)
        pos.append(po)
        wl, wr = p["Wl"][:, po], p["Wr"][:, po]
        if inperm is not None:
            wl, wr = wl[inperm, :], wr[inperm, :]
        xls.append(h @ wl + p["bl"][po])
        xrs.append(h @ wr + p["br"][po])
        wes.append(p["We"][:, po])
        atts.append(p["att"].transpose(1, 0).reshape(cc))
    sum_c = sum(g * 16 for g in gs)
    pad = jnp.zeros((_TAB - _N, sum_c), jnp.float32)
    xlt = jnp.concatenate([jnp.concatenate(xls, axis=1), pad], axis=0)
    xrt = jnp.concatenate([jnp.concatenate(xrs, axis=1), pad], axis=0)
    wet = jnp.concatenate(wes, axis=1)
    att = jnp.concatenate(atts)
    e_main = ea @ wet
    e_tail = jnp.broadcast_to(ea_mean @ wet, (_B, sum_c))
    ek = _edge_k4 if len(gs) == 1 else _edge_k11
    parts = ek(xlt, xrt, ei, e_main, tidx, e_tail, att)
    acc = parts.reshape(2, _ACC_ROWS, -1).sum(0)[:_N]
    outs = []
    for t, g in enumerate(gs):
        cc = g * 16
        off = sum(gg * 16 for gg in gs[:t])
        num = acc[:, off:off + cc]
        den = acc[:, sum_c + t * 16: sum_c + (t + 1) * 16]
        outs.append(num / (jnp.tile(den, (1, g)) + 1e-16))
    return outs, pos


def _bn(h, g, b):
    mu = h.mean(0)
    var = h.var(0)
    return (h - mu) / jnp.sqrt(var + 1e-5) * g + b


def kernel(x, edge_index, edge_attr, mask, batch, params):
    ei = edge_index.astype(jnp.int32)
    tidx = jnp.concatenate([
        jnp.arange(_N, dtype=jnp.int32),
        jnp.full((_TAILN - _N,), _N, jnp.int32),
    ])
    ea_mean = edge_attr.mean(0)
    p = params

    args = (ei, edge_attr, ea_mean, tidx)
    (h,), (po,) = _edge_pass([p["init"]], (4,), x, *args, None)
    h = h + p["init"]["bias"][po]
    h = jax.nn.leaky_relu(_bn(h, p["bn0_g"][po], p["bn0_b"][po]), 0.01)
    (h,), (po2,) = _edge_pass([p["hid"]], (4,), h, *args, po)
    h = h + p["hid"]["bias"][po2]
    h = jax.nn.leaky_relu(_bn(h, p["bn1_g"][po2], p["bn1_b"][po2]), 0.01)
    (om, osat), _ = _edge_pass([p["mus"], p["sat"]], (1, 1), h, *args, po2)
    mus = om.mean(axis=1) + p["mus"]["bias"][0]
    satn = osat.mean(axis=1) + p["sat"]["bias"][0]

    w = (mask == 0).astype(jnp.float32)
    batch_p = jnp.pad(batch.astype(jnp.int32), (0, _NP - _N),
                      constant_values=_NG)
    oh = (batch_p[:, None] == jnp.arange(128, dtype=jnp.int32)[None, :]
          ).astype(jnp.float32)
    lhs = jnp.pad(jnp.stack([satn * w, w]), ((0, 6), (0, _NP - _N)))
    pooled = _pool(lhs, oh)
    sat = pooled[0, :_NG] / jnp.maximum(pooled[1, :_NG], 1.0)
    return (mus, sat)
